# Initial kernel scaffold; baseline (speedup 1.0000x reference)
import jax
import jax.numpy as jnp
from jax.experimental import pallas as pl


def kernel(x, edge_index, edge_attr, rev_edge_index, W_i, b_i, W_h, b_h, W_o, b_o, W_nt, b_nt):
    raise NotImplementedError("write your pallas kernel here")


# speedup vs baseline: 1.8480x; 1.8480x over previous
import functools

import jax
import jax.numpy as jnp
from jax import lax
from jax.experimental import pallas as pl
from jax.experimental.pallas import tpu as pltpu
from jax.experimental.pallas import tpu_sc as plsc

N_NODES = 10000
N_EDGES = 160000
D_NODE = 256
D_EDGE = 16
HIDDEN = 256
DEPTH = 5

NC = 2
NS = 16
NW = NC * NS
CH = 128
NCH = N_EDGES // CH
HHALF = HIDDEN // 2
ZR = 80
NZCH = N_NODES // ZR

_f32 = jnp.float32


def _sc_mesh():
    return plsc.VectorSubcoreMesh(core_axis_name="c", subcore_axis_name="s")


def _gather1_body(tab, idx, out, idx_v, rows_v, sem):
    c = lax.axis_index("c")
    s = lax.axis_index("s")
    wid = s * NC + c
    nj = (NCH + NW - 1) // NW

    def body(j, carry):
        cid = wid + j * NW

        @pl.when(cid < NCH)
        def _():
            base = cid * CH
            pltpu.sync_copy(idx.at[pl.ds(base, CH)], idx_v)
            pltpu.async_copy(tab.at[idx_v], rows_v, sem).wait()
            pltpu.sync_copy(rows_v, out.at[pl.ds(base, CH)])

        return carry

    lax.fori_loop(0, nj, body, 0)


def _sc_gather(tab, idx):
    rows, d = tab.shape
    e = idx.shape[0]
    assert e % CH == 0
    return pl.kernel(
        _gather1_body,
        out_type=jax.ShapeDtypeStruct((e, d), tab.dtype),
        mesh=_sc_mesh(),
        scratch_types=[
            pltpu.VMEM((CH,), jnp.int32),
            pltpu.VMEM((CH, d), tab.dtype),
            pltpu.SemaphoreType.DMA,
        ],
    )(tab, idx)


def _gather2_body(s_tab, p_tab, src, rev, o1, o2, i1_v, i2_v, r1_v, r2_v, sem1, sem2):
    c = lax.axis_index("c")
    s = lax.axis_index("s")
    wid = s * NC + c
    nj = (NCH + NW - 1) // NW

    def body(j, carry):
        cid = wid + j * NW

        @pl.when(cid < NCH)
        def _():
            base = cid * CH
            pltpu.sync_copy(src.at[pl.ds(base, CH)], i1_v)
            pltpu.sync_copy(rev.at[pl.ds(base, CH)], i2_v)
            d1 = pltpu.async_copy(s_tab.at[i1_v], r1_v, sem1)
            d2 = pltpu.async_copy(p_tab.at[i2_v], r2_v, sem2)
            d1.wait()
            d2.wait()
            pltpu.sync_copy(r1_v, o1.at[pl.ds(base, CH)])
            pltpu.sync_copy(r2_v, o2.at[pl.ds(base, CH)])

        return carry

    lax.fori_loop(0, nj, body, 0)


def _sc_gather2(s_tab, p_tab, src, rev):
    e = src.shape[0]
    d = s_tab.shape[1]
    return pl.kernel(
        _gather2_body,
        out_type=(
            jax.ShapeDtypeStruct((e, d), _f32),
            jax.ShapeDtypeStruct((e, d), _f32),
        ),
        mesh=_sc_mesh(),
        scratch_types=[
            pltpu.VMEM((CH,), jnp.int32),
            pltpu.VMEM((CH,), jnp.int32),
            pltpu.VMEM((CH, d), _f32),
            pltpu.VMEM((CH, d), _f32),
            pltpu.SemaphoreType.DMA,
            pltpu.SemaphoreType.DMA,
        ],
    )(s_tab, p_tab, src, rev)


def _scatter_body(p, dst, out, idx_v, rows_v, zb, acc):
    c = lax.axis_index("c")
    s = lax.axis_index("s")

    def zbody(i, carry):
        r = i // (HHALF // 16)
        k = i % (HHALF // 16)
        zb[r, pl.ds(k * 16, 16)] = jnp.zeros((16,), _f32)
        return carry

    lax.fori_loop(0, ZR * (HHALF // 16), zbody, 0)

    nz = (NZCH + NS - 1) // NS

    def z2(i, carry):
        cid = s + i * NS

        @pl.when(cid < NZCH)
        def _():
            pltpu.sync_copy(zb, acc.at[pl.ds(cid * ZR, ZR)])

        return carry

    lax.fori_loop(0, nz, z2, 0)
    plsc.subcore_barrier()

    nj = (NCH + NS - 1) // NS

    def body(j, carry):
        cid = s + j * NS

        @pl.when(cid < NCH)
        def _():
            base = cid * CH
            pltpu.sync_copy(dst.at[pl.ds(base, CH)], idx_v)
            pltpu.sync_copy(p.at[pl.ds(base, CH), pl.ds(c * HHALF, HHALF)], rows_v)
            pltpu.sync_copy(rows_v, acc.at[idx_v], add=True)

        return carry

    lax.fori_loop(0, nj, body, 0)
    plsc.subcore_barrier()

    def wr(i, carry):
        cid = s + i * NS

        @pl.when(cid < NZCH)
        def _():
            r = cid * ZR
            pltpu.sync_copy(acc.at[pl.ds(r, ZR)], out.at[pl.ds(r, ZR), pl.ds(c * HHALF, HHALF)])

        return carry

    lax.fori_loop(0, nz, wr, 0)


def _sc_segment_sum(p, dst):
    return pl.kernel(
        _scatter_body,
        out_type=jax.ShapeDtypeStruct((N_NODES, HIDDEN), _f32),
        mesh=_sc_mesh(),
        scratch_types=[
            pltpu.VMEM((CH,), jnp.int32),
            pltpu.VMEM((CH, HHALF), _f32),
            pltpu.VMEM((ZR, HHALF), _f32),
            pltpu.VMEM_SHARED((N_NODES, HHALF), _f32),
        ],
    )(p, dst)


BN = 2000
BE = 2000


def _pre_body(x_ref, w1t_ref, wntt_ref, bnt_ref, xw1_ref, tx_ref):
    xb = x_ref[...]
    xw1_ref[...] = jnp.dot(xb, w1t_ref[...], preferred_element_type=_f32)
    tx_ref[...] = jnp.dot(xb, wntt_ref[...], preferred_element_type=_f32) + bnt_ref[...]


def _tc_pre(x, w1t, wntt, bnt):
    n = x.shape[0]
    grid = (n // BN,)
    return pl.pallas_call(
        _pre_body,
        grid=grid,
        in_specs=[
            pl.BlockSpec((BN, D_NODE), lambda i: (i, 0)),
            pl.BlockSpec((D_NODE, HIDDEN), lambda i: (0, 0)),
            pl.BlockSpec((D_NODE, HIDDEN), lambda i: (0, 0)),
            pl.BlockSpec((1, HIDDEN), lambda i: (0, 0)),
        ],
        out_specs=(
            pl.BlockSpec((BN, HIDDEN), lambda i: (i, 0)),
            pl.BlockSpec((BN, HIDDEN), lambda i: (i, 0)),
        ),
        out_shape=(
            jax.ShapeDtypeStruct((n, HIDDEN), _f32),
            jax.ShapeDtypeStruct((n, HIDDEN), _f32),
        ),
    )(x, w1t, wntt, bnt)


def _mm0_body(gx_ref, ea_ref, w2t_ref, bi_ref, wht_ref, a_ref, p_ref):
    a = gx_ref[...] + jnp.dot(ea_ref[...], w2t_ref[...], preferred_element_type=_f32) + bi_ref[...]
    a_ref[...] = a
    p_ref[...] = jnp.dot(jnp.maximum(a, 0.0), wht_ref[...], preferred_element_type=_f32)


def _tc_mm0(gx, ea, w2t, bi, wht):
    e = gx.shape[0]
    grid = (e // BE,)
    return pl.pallas_call(
        _mm0_body,
        grid=grid,
        in_specs=[
            pl.BlockSpec((BE, HIDDEN), lambda i: (i, 0)),
            pl.BlockSpec((BE, D_EDGE), lambda i: (i, 0)),
            pl.BlockSpec((D_EDGE, HIDDEN), lambda i: (0, 0)),
            pl.BlockSpec((1, HIDDEN), lambda i: (0, 0)),
            pl.BlockSpec((HIDDEN, HIDDEN), lambda i: (0, 0)),
        ],
        out_specs=(
            pl.BlockSpec((BE, HIDDEN), lambda i: (i, 0)),
            pl.BlockSpec((BE, HIDDEN), lambda i: (i, 0)),
        ),
        out_shape=(
            jax.ShapeDtypeStruct((e, HIDDEN), _f32),
            jax.ShapeDtypeStruct((e, HIDDEN), _f32),
        ),
    )(gx, ea, w2t, bi, wht)


def _mm_body(a_ref, ss_ref, pr_ref, bh_ref, wht_ref, p_ref):
    g = a_ref[...] + ss_ref[...] - pr_ref[...] + bh_ref[...]
    p_ref[...] = jnp.dot(jnp.maximum(g, 0.0), wht_ref[...], preferred_element_type=_f32)


def _tc_mm(a, ssrc, prev, bh, wht):
    e = a.shape[0]
    grid = (e // BE,)
    return pl.pallas_call(
        _mm_body,
        grid=grid,
        in_specs=[
            pl.BlockSpec((BE, HIDDEN), lambda i: (i, 0)),
            pl.BlockSpec((BE, HIDDEN), lambda i: (i, 0)),
            pl.BlockSpec((BE, HIDDEN), lambda i: (i, 0)),
            pl.BlockSpec((1, HIDDEN), lambda i: (0, 0)),
            pl.BlockSpec((HIDDEN, HIDDEN), lambda i: (0, 0)),
        ],
        out_specs=pl.BlockSpec((BE, HIDDEN), lambda i: (i, 0)),
        out_shape=jax.ShapeDtypeStruct((e, HIDDEN), _f32),
    )(a, ssrc, prev, bh, wht)


def _ew_body(a_ref, ss_ref, pr_ref, bh_ref, h_ref):
    g = a_ref[...] + ss_ref[...] - pr_ref[...] + bh_ref[...]
    h_ref[...] = jnp.maximum(g, 0.0)


def _tc_ew(a, ssrc, prev, bh):
    e = a.shape[0]
    grid = (e // BE,)
    return pl.pallas_call(
        _ew_body,
        grid=grid,
        in_specs=[
            pl.BlockSpec((BE, HIDDEN), lambda i: (i, 0)),
            pl.BlockSpec((BE, HIDDEN), lambda i: (i, 0)),
            pl.BlockSpec((BE, HIDDEN), lambda i: (i, 0)),
            pl.BlockSpec((1, HIDDEN), lambda i: (0, 0)),
        ],
        out_specs=pl.BlockSpec((BE, HIDDEN), lambda i: (i, 0)),
        out_shape=jax.ShapeDtypeStruct((e, HIDDEN), _f32),
    )(a, ssrc, prev, bh)


def _final_body(x_ref, sn_ref, tx_ref, wo1t_ref, wo2t_ref, bo_ref, out_ref):
    sn = sn_ref[...]
    rs = jnp.sum(sn, axis=1, keepdims=True)
    mn = jnp.where(rs == 0.0, tx_ref[...], sn)
    acc = jnp.dot(x_ref[...], wo1t_ref[...], preferred_element_type=_f32)
    acc = acc + jnp.dot(mn, wo2t_ref[...], preferred_element_type=_f32) + bo_ref[...]
    out_ref[...] = jnp.maximum(acc, 0.0)


def _tc_final(x, snode, tx, wo1t, wo2t, bo):
    n = x.shape[0]
    grid = (n // BN,)
    return pl.pallas_call(
        _final_body,
        grid=grid,
        in_specs=[
            pl.BlockSpec((BN, D_NODE), lambda i: (i, 0)),
            pl.BlockSpec((BN, HIDDEN), lambda i: (i, 0)),
            pl.BlockSpec((BN, HIDDEN), lambda i: (i, 0)),
            pl.BlockSpec((D_NODE, HIDDEN), lambda i: (0, 0)),
            pl.BlockSpec((HIDDEN, HIDDEN), lambda i: (0, 0)),
            pl.BlockSpec((1, HIDDEN), lambda i: (0, 0)),
        ],
        out_specs=pl.BlockSpec((BN, HIDDEN), lambda i: (i, 0)),
        out_shape=jax.ShapeDtypeStruct((n, HIDDEN), _f32),
    )(x, snode, tx, wo1t, wo2t, bo)


def kernel(x, edge_index, edge_attr, rev_edge_index, W_i, b_i, W_h, b_h, W_o, b_o, W_nt, b_nt):
    src = edge_index[0].astype(jnp.int32)
    dst = edge_index[1].astype(jnp.int32)
    rev = rev_edge_index.astype(jnp.int32)

    w1t = W_i[:, :D_NODE].T
    w2t = W_i[:, D_NODE:].T
    wht = W_h.T
    wntt = W_nt.T
    wo1t = W_o[:, :D_NODE].T
    wo2t = W_o[:, D_NODE:].T
    bi = b_i.reshape(1, -1)
    bh = b_h.reshape(1, -1)
    bo = b_o.reshape(1, -1)
    bnt = b_nt.reshape(1, -1)

    xw1, tx = _tc_pre(x, w1t, wntt, bnt)
    gx = _sc_gather(xw1, src)
    a, p = _tc_mm0(gx, edge_attr, w2t, bi, wht)

    for t in range(1, DEPTH):
        s_nodes = _sc_segment_sum(p, dst)
        ssrc, prev = _sc_gather2(s_nodes, p, src, rev)
        if t < DEPTH - 1:
            p = _tc_mm(a, ssrc, prev, bh, wht)
        else:
            h = _tc_ew(a, ssrc, prev, bh)

    snode = _sc_segment_sum(h, dst)
    return _tc_final(x, snode, tx, wo1t, wo2t, bo)
